# Initial kernel scaffold; baseline (speedup 1.0000x reference)
import jax
import jax.numpy as jnp
from jax.experimental import pallas as pl


def kernel(x, W1, b1, gamma, beta, W2, b2):
    raise NotImplementedError("write your pallas kernel here")


# speedup vs baseline: 2.3866x; 2.3866x over previous
import functools

import jax
import jax.numpy as jnp
from jax.experimental import pallas as pl
from jax.experimental.pallas import tpu as pltpu

TOPK = 8
EPS = 1e-5
_NEG = -3.0e38


def _pass1_kernel(x_ref, w1_ref, b1_ref, h_ref, stats_ref):
    h = jnp.dot(x_ref[...], w1_ref[...], preferred_element_type=jnp.float32)
    h = jnp.maximum(h + b1_ref[...], 0.0)
    h_ref[...] = h
    s = jnp.sum(h, axis=0, keepdims=True)
    ss = jnp.sum(h * h, axis=0, keepdims=True)
    pad = jnp.zeros((6, h.shape[1]), jnp.float32)
    stats_ref[...] = jnp.concatenate([s, ss, pad], axis=0)[None]


def _pass2_kernel(h_ref, stats_ref, w2_ref, gamma_ref, beta_ref, b2_ref,
                  out_ref, *, inv_b):
    stats = stats_ref[...]
    s = jnp.sum(stats[:, 0, :], axis=0)
    ss = jnp.sum(stats[:, 1, :], axis=0)
    mean = s * inv_b
    var = jnp.maximum(ss * inv_b - mean * mean, 0.0)
    scale = gamma_ref[0] * jax.lax.rsqrt(var + EPS)
    shift = beta_ref[0] - mean * scale
    hn = h_ref[...] * scale[None, :] + shift[None, :]
    logits = jnp.dot(hn.astype(jnp.bfloat16), w2_ref[...],
                     preferred_element_type=jnp.float32) + b2_ref[...]

    ncols = logits.shape[1]
    iota = jax.lax.broadcasted_iota(jnp.int32, logits.shape, 1)
    work = logits
    sel = jnp.zeros(logits.shape, jnp.bool_)
    for _ in range(TOPK):
        m = jnp.max(work, axis=1, keepdims=True)
        cand = work >= m
        first = jnp.min(jnp.where(cand, iota, ncols), axis=1, keepdims=True)
        pick = iota == first
        sel = jnp.logical_or(sel, pick)
        work = jnp.where(pick, _NEG, work)

    rowmax = jnp.max(logits, axis=1, keepdims=True)
    p = jnp.where(sel, jnp.exp(logits - rowmax), 0.0)
    out_ref[...] = p / jnp.sum(p, axis=1, keepdims=True)


def kernel(x, W1, b1, gamma, beta, W2, b2):
    B, D = x.shape
    H = W1.shape[1]
    E = W2.shape[1]
    BB1 = 512
    BB2 = 512
    nb1 = B // BB1
    nb2 = B // BB2

    xb = x.astype(jnp.bfloat16)
    w1b = W1.astype(jnp.bfloat16)
    w2b = W2.astype(jnp.bfloat16)
    b1r = b1.reshape(1, H)
    gammar = gamma.reshape(1, H)
    betar = beta.reshape(1, H)
    b2r = b2.reshape(1, E)

    h, stats = pl.pallas_call(
        _pass1_kernel,
        grid=(nb1,),
        in_specs=[
            pl.BlockSpec((BB1, D), lambda i: (i, 0)),
            pl.BlockSpec((D, H), lambda i: (0, 0)),
            pl.BlockSpec((1, H), lambda i: (0, 0)),
        ],
        out_specs=[
            pl.BlockSpec((BB1, H), lambda i: (i, 0)),
            pl.BlockSpec((1, 8, H), lambda i: (i, 0, 0)),
        ],
        out_shape=[
            jax.ShapeDtypeStruct((B, H), jnp.float32),
            jax.ShapeDtypeStruct((nb1, 8, H), jnp.float32),
        ],
        compiler_params=pltpu.CompilerParams(
            dimension_semantics=("parallel",)),
    )(xb, w1b, b1r)

    out = pl.pallas_call(
        functools.partial(_pass2_kernel, inv_b=1.0 / B),
        grid=(nb2,),
        in_specs=[
            pl.BlockSpec((BB2, H), lambda i: (i, 0)),
            pl.BlockSpec((nb1, 8, H), lambda i: (0, 0, 0)),
            pl.BlockSpec((H, E), lambda i: (0, 0)),
            pl.BlockSpec((1, H), lambda i: (0, 0)),
            pl.BlockSpec((1, H), lambda i: (0, 0)),
            pl.BlockSpec((1, E), lambda i: (0, 0)),
        ],
        out_specs=pl.BlockSpec((BB2, E), lambda i: (i, 0)),
        out_shape=jax.ShapeDtypeStruct((B, E), jnp.float32),
        compiler_params=pltpu.CompilerParams(
            dimension_semantics=("parallel",)),
    )(h, stats, w2b, gammar, betar, b2r)
    return out


# speedup vs baseline: 3.5001x; 1.4666x over previous
import functools

import jax
import jax.numpy as jnp
from jax.experimental import pallas as pl
from jax.experimental.pallas import tpu as pltpu

TOPK = 8
EPS = 1e-5
_NEG = -3.0e38


def _pass1_kernel(x_ref, w1_ref, b1_ref, h_ref, stats_ref):
    h = jnp.dot(x_ref[...].astype(jnp.bfloat16), w1_ref[...],
                preferred_element_type=jnp.float32)
    h = jnp.maximum(h + b1_ref[...], 0.0)
    h_ref[...] = h
    s = jnp.sum(h, axis=0, keepdims=True)
    ss = jnp.sum(h * h, axis=0, keepdims=True)
    pad = jnp.zeros((6, h.shape[1]), jnp.float32)
    stats_ref[...] = jnp.concatenate([s, ss, pad], axis=0)[None]


def _pass2_kernel(h_ref, stats_ref, w2_ref, gamma_ref, beta_ref, b2_ref,
                  out_ref, *, inv_b):
    stats = stats_ref[...]
    s = jnp.sum(stats[:, 0, :], axis=0)
    ss = jnp.sum(stats[:, 1, :], axis=0)
    mean = s * inv_b
    var = jnp.maximum(ss * inv_b - mean * mean, 0.0)
    scale = gamma_ref[0] * jax.lax.rsqrt(var + EPS)
    shift = beta_ref[0] - mean * scale
    hn = h_ref[...] * scale[None, :] + shift[None, :]
    logits = jnp.dot(hn.astype(jnp.bfloat16), w2_ref[...],
                     preferred_element_type=jnp.float32) + b2_ref[...]

    ncols = logits.shape[1]
    iota = jax.lax.broadcasted_iota(jnp.int32, logits.shape, 1)
    work = logits
    sel = jnp.zeros(logits.shape, jnp.bool_)
    for _ in range(TOPK):
        m = jnp.max(work, axis=1, keepdims=True)
        cand = work >= m
        first = jnp.min(jnp.where(cand, iota, ncols), axis=1, keepdims=True)
        pick = iota == first
        sel = jnp.logical_or(sel, pick)
        work = jnp.where(pick, _NEG, work)

    rowmax = jnp.max(logits, axis=1, keepdims=True)
    p = jnp.where(sel, jnp.exp(logits - rowmax), 0.0)
    out_ref[...] = p / jnp.sum(p, axis=1, keepdims=True)


def kernel(x, W1, b1, gamma, beta, W2, b2):
    B, D = x.shape
    H = W1.shape[1]
    E = W2.shape[1]
    BB1 = 512
    BB2 = 512
    nb1 = B // BB1
    nb2 = B // BB2

    w1b = W1.astype(jnp.bfloat16)
    w2b = W2.astype(jnp.bfloat16)
    b1r = b1.reshape(1, H)
    gammar = gamma.reshape(1, H)
    betar = beta.reshape(1, H)
    b2r = b2.reshape(1, E)

    h, stats = pl.pallas_call(
        _pass1_kernel,
        grid=(nb1,),
        in_specs=[
            pl.BlockSpec((BB1, D), lambda i: (i, 0)),
            pl.BlockSpec((D, H), lambda i: (0, 0)),
            pl.BlockSpec((1, H), lambda i: (0, 0)),
        ],
        out_specs=[
            pl.BlockSpec((BB1, H), lambda i: (i, 0)),
            pl.BlockSpec((1, 8, H), lambda i: (i, 0, 0)),
        ],
        out_shape=[
            jax.ShapeDtypeStruct((B, H), jnp.float32),
            jax.ShapeDtypeStruct((nb1, 8, H), jnp.float32),
        ],
        compiler_params=pltpu.CompilerParams(
            dimension_semantics=("parallel",)),
    )(x, w1b, b1r)

    out = pl.pallas_call(
        functools.partial(_pass2_kernel, inv_b=1.0 / B),
        grid=(nb2,),
        in_specs=[
            pl.BlockSpec((BB2, H), lambda i: (i, 0)),
            pl.BlockSpec((nb1, 8, H), lambda i: (0, 0, 0)),
            pl.BlockSpec((H, E), lambda i: (0, 0)),
            pl.BlockSpec((1, H), lambda i: (0, 0)),
            pl.BlockSpec((1, H), lambda i: (0, 0)),
            pl.BlockSpec((1, E), lambda i: (0, 0)),
        ],
        out_specs=pl.BlockSpec((BB2, E), lambda i: (i, 0)),
        out_shape=jax.ShapeDtypeStruct((B, E), jnp.float32),
        compiler_params=pltpu.CompilerParams(
            dimension_semantics=("parallel",)),
    )(h, stats, w2b, gammar, betar, b2r)
    return out


# speedup vs baseline: 4.3190x; 1.2340x over previous
import functools

import jax
import jax.numpy as jnp
from jax.experimental import pallas as pl
from jax.experimental.pallas import tpu as pltpu

TOPK = 8
EPS = 1e-5
_NEG = -3.0e38


def _pass1_kernel(x_ref, w1_ref, b1_ref, ht_ref, stats_ref):
    h = jnp.dot(x_ref[...].astype(jnp.bfloat16), w1_ref[...],
                preferred_element_type=jnp.float32)
    h = jnp.maximum(h + b1_ref[...], 0.0)
    ht_ref[...] = h.T
    s = jnp.sum(h, axis=0, keepdims=True)
    ss = jnp.sum(h * h, axis=0, keepdims=True)
    pad = jnp.zeros((6, h.shape[1]), jnp.float32)
    stats_ref[...] = jnp.concatenate([s, ss, pad], axis=0)[None]


def _pass2_kernel(ht_ref, stats_ref, w2t_ref, gamma_ref, beta_ref, b2_ref,
                  out_ref, *, inv_b):
    stats = stats_ref[...]
    s = jnp.sum(stats[:, 0, :], axis=0)
    ss = jnp.sum(stats[:, 1, :], axis=0)
    mean = s * inv_b
    var = jnp.maximum(ss * inv_b - mean * mean, 0.0)
    scale = gamma_ref[0] * jax.lax.rsqrt(var + EPS)
    shift = beta_ref[0] - mean * scale
    hn = ht_ref[...] * scale[:, None] + shift[:, None]
    logits = jnp.dot(w2t_ref[...], hn.astype(jnp.bfloat16),
                     preferred_element_type=jnp.float32) + b2_ref[...]

    nexp = logits.shape[0]
    iota = jax.lax.broadcasted_iota(
        jnp.int32, logits.shape, 0).astype(jnp.float32)
    work = logits
    sel = jnp.zeros(logits.shape, jnp.bool_)
    for _ in range(TOPK):
        m = jnp.max(work, axis=0, keepdims=True)
        cand = work >= m
        first = jnp.min(jnp.where(cand, iota, float(nexp)),
                        axis=0, keepdims=True)
        pick = iota == first
        sel = jnp.logical_or(sel, pick)
        work = jnp.where(pick, _NEG, work)

    rowmax = jnp.max(logits, axis=0, keepdims=True)
    p = jnp.where(sel, jnp.exp(logits - rowmax), 0.0)
    out_ref[...] = (p / jnp.sum(p, axis=0, keepdims=True)).T


def kernel(x, W1, b1, gamma, beta, W2, b2):
    B, D = x.shape
    H = W1.shape[1]
    E = W2.shape[1]
    BB1 = 512
    BB2 = 512
    nb1 = B // BB1
    nb2 = B // BB2

    w1b = W1.astype(jnp.bfloat16)
    w2tb = W2.T.astype(jnp.bfloat16)
    b1r = b1.reshape(1, H)
    gammar = gamma.reshape(1, H)
    betar = beta.reshape(1, H)
    b2c = b2.reshape(E, 1)

    ht, stats = pl.pallas_call(
        _pass1_kernel,
        grid=(nb1,),
        in_specs=[
            pl.BlockSpec((BB1, D), lambda i: (i, 0)),
            pl.BlockSpec((D, H), lambda i: (0, 0)),
            pl.BlockSpec((1, H), lambda i: (0, 0)),
        ],
        out_specs=[
            pl.BlockSpec((H, BB1), lambda i: (0, i)),
            pl.BlockSpec((1, 8, H), lambda i: (i, 0, 0)),
        ],
        out_shape=[
            jax.ShapeDtypeStruct((H, B), jnp.float32),
            jax.ShapeDtypeStruct((nb1, 8, H), jnp.float32),
        ],
        compiler_params=pltpu.CompilerParams(
            dimension_semantics=("parallel",)),
    )(x, w1b, b1r)

    out = pl.pallas_call(
        functools.partial(_pass2_kernel, inv_b=1.0 / B),
        grid=(nb2,),
        in_specs=[
            pl.BlockSpec((H, BB2), lambda i: (0, i)),
            pl.BlockSpec((nb1, 8, H), lambda i: (0, 0, 0)),
            pl.BlockSpec((E, H), lambda i: (0, 0)),
            pl.BlockSpec((1, H), lambda i: (0, 0)),
            pl.BlockSpec((1, H), lambda i: (0, 0)),
            pl.BlockSpec((E, 1), lambda i: (0, 0)),
        ],
        out_specs=pl.BlockSpec((BB2, E), lambda i: (i, 0)),
        out_shape=jax.ShapeDtypeStruct((B, E), jnp.float32),
        compiler_params=pltpu.CompilerParams(
            dimension_semantics=("parallel",)),
    )(ht, stats, w2tb, gammar, betar, b2c)
    return out


# speedup vs baseline: 4.9424x; 1.1443x over previous
import functools

import jax
import jax.numpy as jnp
from jax.experimental import pallas as pl
from jax.experimental.pallas import tpu as pltpu

TOPK = 8
EPS = 1e-5
_NEG = -3.0e38


def _fused_kernel(x_ref, w1_ref, b1_ref, gamma_ref, beta_ref, w2_ref, b2_ref,
                  out_ref, ht_ref, w1b_ref, stats_ref, *, nb, bb, inv_b):
    i = pl.program_id(0)

    @pl.when(i == 0)
    def _():
        w1b_ref[...] = w1_ref[...].astype(jnp.bfloat16)

    @pl.when(i < nb)
    def _phase1():
        h = jnp.dot(x_ref[...].astype(jnp.bfloat16), w1b_ref[...],
                    preferred_element_type=jnp.float32)
        h = jnp.maximum(h + b1_ref[...], 0.0)
        ht_ref[:, pl.ds(i * bb, bb)] = h.T
        s = jnp.sum(h, axis=0, keepdims=True)
        ss = jnp.sum(h * h, axis=0, keepdims=True)

        @pl.when(i == 0)
        def _():
            stats_ref[0:1, :] = s
            stats_ref[1:2, :] = ss

        @pl.when(i > 0)
        def _():
            stats_ref[0:1, :] = stats_ref[0:1, :] + s
            stats_ref[1:2, :] = stats_ref[1:2, :] + ss

    @pl.when(i >= nb)
    def _phase2():
        j = i - nb
        mean = stats_ref[0, :] * inv_b
        var = jnp.maximum(stats_ref[1, :] * inv_b - mean * mean, 0.0)
        scale = gamma_ref[0] * jax.lax.rsqrt(var + EPS)
        shift = beta_ref[0] - mean * scale
        ht = ht_ref[:, pl.ds(j * bb, bb)]
        hn = ht * scale[:, None] + shift[:, None]
        w2b = w2_ref[...].astype(jnp.bfloat16)
        logits = jax.lax.dot_general(
            w2b, hn.astype(jnp.bfloat16), (((0,), (0,)), ((), ())),
            preferred_element_type=jnp.float32) + b2_ref[...]

        nexp = logits.shape[0]
        iota = jax.lax.broadcasted_iota(
            jnp.int32, logits.shape, 0).astype(jnp.float32)
        work = logits
        sel = jnp.zeros(logits.shape, jnp.bool_)
        for _ in range(TOPK):
            m = jnp.max(work, axis=0, keepdims=True)
            cand = work >= m
            first = jnp.min(jnp.where(cand, iota, float(nexp)),
                            axis=0, keepdims=True)
            pick = iota == first
            sel = jnp.logical_or(sel, pick)
            work = jnp.where(pick, _NEG, work)

        rowmax = jnp.max(logits, axis=0, keepdims=True)
        p = jnp.where(sel, jnp.exp(logits - rowmax), 0.0)
        out_ref[...] = (p / jnp.sum(p, axis=0, keepdims=True)).T


def kernel(x, W1, b1, gamma, beta, W2, b2):
    B, D = x.shape
    H = W1.shape[1]
    E = W2.shape[1]
    BB = 512
    nb = B // BB

    b1r = b1.reshape(1, H)
    gammar = gamma.reshape(1, H)
    betar = beta.reshape(1, H)
    b2c = b2.reshape(E, 1)

    out = pl.pallas_call(
        functools.partial(_fused_kernel, nb=nb, bb=BB, inv_b=1.0 / B),
        grid=(2 * nb,),
        in_specs=[
            pl.BlockSpec((BB, D), lambda i: (jnp.minimum(i, nb - 1), 0)),
            pl.BlockSpec((D, H), lambda i: (0, 0)),
            pl.BlockSpec((1, H), lambda i: (0, 0)),
            pl.BlockSpec((1, H), lambda i: (0, 0)),
            pl.BlockSpec((1, H), lambda i: (0, 0)),
            pl.BlockSpec((H, E), lambda i: (0, 0)),
            pl.BlockSpec((E, 1), lambda i: (0, 0)),
        ],
        out_specs=pl.BlockSpec((BB, E), lambda i: (jnp.maximum(i - nb, 0), 0)),
        out_shape=jax.ShapeDtypeStruct((B, E), jnp.float32),
        scratch_shapes=[
            pltpu.VMEM((H, B), jnp.float32),
            pltpu.VMEM((D, H), jnp.bfloat16),
            pltpu.VMEM((8, H), jnp.float32),
        ],
        compiler_params=pltpu.CompilerParams(
            dimension_semantics=("arbitrary",)),
    )(x, W1, b1r, gammar, betar, W2, b2c)
    return out
